# baseline (speedup 1.0000x reference)
import jax
import jax.numpy as jnp
from jax import lax
from jax.experimental import pallas as pl
from jax.experimental.pallas import tpu as pltpu
from jax.experimental.pallas import tpu_sc as plsc

D_MODEL = 768
BATCHES = 4
SEQ_LEN = 8192
ROWS_PER_TILE = 256
GROUP_ROWS = 16
NUM_GROUPS = ROWS_PER_TILE // GROUP_ROWS
LANE_GROUPS = D_MODEL // 16


def _sc_body(x_hbm, emb_hbm, out_hbm,
             xv000, xv001, xv002, xv003,
             xv100, xv101, xv102, xv103,
             ev0, ev1,
             esem0, esem1, isem0, isem1, osem0, osem1):
    xv = ((xv000, xv001, xv002, xv003), (xv100, xv101, xv102, xv103))
    ev = (ev0, ev1)
    esem = (esem0, esem1)
    isem = (isem0, isem1)
    osem = (osem0, osem1)

    wid = lax.axis_index("c") * 16 + lax.axis_index("s")
    row_base = wid * ROWS_PER_TILE

    def issue_loads(e, sl):
        r0 = pl.multiple_of(row_base + e * GROUP_ROWS, GROUP_ROWS)
        pltpu.async_copy(emb_hbm.at[pl.ds(r0, GROUP_ROWS), :], ev[sl],
                         esem[sl])
        for b in range(BATCHES):
            pltpu.async_copy(x_hbm.at[b, pl.ds(r0, GROUP_ROWS), :],
                             xv[sl][b], isem[sl])

    def issue_stores(e, sl):
        r0 = pl.multiple_of(row_base + e * GROUP_ROWS, GROUP_ROWS)
        for b in range(BATCHES):
            pltpu.async_copy(xv[sl][b],
                             out_hbm.at[b, pl.ds(r0, GROUP_ROWS), :],
                             osem[sl])

    def wait(sem, dst, n):
        for _ in range(n):
            pltpu.make_async_copy(x_hbm.at[0, pl.ds(0, GROUP_ROWS), :], dst,
                                  sem).wait()

    def compute(sl):
        bufs = xv[sl]

        def row(r, c1):
            for j in range(LANE_GROUPS):
                s16 = pl.ds(j * 16, 16)
                e_val = ev[sl][r, s16]
                for b in range(BATCHES):
                    bufs[b][r, s16] = bufs[b][r, s16] + e_val
            return c1

        lax.fori_loop(0, GROUP_ROWS, row, 0)

    def group(e, sl, first_pair, last):
        other = 1 - sl
        if not first_pair:
            wait(osem[other], xv[other][0], BATCHES)
        if not last:
            issue_loads(e + 1, other)
        wait(esem[sl], ev[sl], 1)
        wait(isem[sl], xv[sl][0], BATCHES)
        compute(sl)
        issue_stores(e, sl)

    issue_loads(0, 0)
    group(0, 0, True, False)
    group(1, 1, True, False)

    def pair(k, carry):
        group(2 * k, 0, False, False)
        group(2 * k + 1, 1, False, False)
        return carry

    lax.fori_loop(1, NUM_GROUPS // 2 - 1, pair, 0)
    group(NUM_GROUPS - 2, 0, False, False)
    group(NUM_GROUPS - 1, 1, False, True)
    wait(osem[0], xv[0][0], BATCHES)
    wait(osem[1], xv[1][0], BATCHES)


def kernel(x, emb):
    bs, sl, d = x.shape
    mesh = plsc.VectorSubcoreMesh(core_axis_name="c", subcore_axis_name="s")
    buf = pltpu.VMEM((GROUP_ROWS, D_MODEL), jnp.float32)
    return pl.kernel(
        _sc_body,
        out_type=jax.ShapeDtypeStruct((bs, sl, d), x.dtype),
        mesh=mesh,
        scratch_types=[buf] * 10 + [pltpu.SemaphoreType.DMA] * 6,
        compiler_params=pltpu.CompilerParams(use_tc_tiling_on_sc=True),
    )(x, emb)


# speedup vs baseline: 1.2210x; 1.2210x over previous
import jax
import jax.numpy as jnp
from jax import lax
from jax.experimental import pallas as pl
from jax.experimental.pallas import tpu as pltpu
from jax.experimental.pallas import tpu_sc as plsc

D_MODEL = 768
BATCHES = 4
SEQ_LEN = 8192
HEAD = 6144
GROUP_ROWS = 16
LANE_GROUPS = D_MODEL // 16
SEQ_BLOCK = 1024
ROWS_PER_TILE = (SEQ_LEN - HEAD) // 32
NUM_GROUPS = ROWS_PER_TILE // GROUP_ROWS


def _sc_body(out_hbm, x_hbm, emb_hbm,
             xv000, xv001, xv002, xv003,
             xv100, xv101, xv102, xv103,
             ev0, ev1,
             esem0, esem1, isem0, isem1, osem0, osem1):
    xv = ((xv000, xv001, xv002, xv003), (xv100, xv101, xv102, xv103))
    ev = (ev0, ev1)
    esem = (esem0, esem1)
    isem = (isem0, isem1)
    osem = (osem0, osem1)

    wid = lax.axis_index("c") * 16 + lax.axis_index("s")
    row_base = HEAD + wid * ROWS_PER_TILE

    def issue_loads(e, sl):
        r0 = pl.multiple_of(row_base + e * GROUP_ROWS, GROUP_ROWS)
        pltpu.async_copy(emb_hbm.at[pl.ds(r0, GROUP_ROWS), :], ev[sl],
                         esem[sl])
        for b in range(BATCHES):
            pltpu.async_copy(x_hbm.at[b, pl.ds(r0, GROUP_ROWS), :],
                             xv[sl][b], isem[sl])

    def issue_stores(e, sl):
        r0 = pl.multiple_of(row_base + e * GROUP_ROWS, GROUP_ROWS)
        for b in range(BATCHES):
            pltpu.async_copy(xv[sl][b],
                             out_hbm.at[b, pl.ds(r0, GROUP_ROWS), :],
                             osem[sl])

    def wait(sem, dst, n):
        for _ in range(n):
            pltpu.make_async_copy(x_hbm.at[0, pl.ds(0, GROUP_ROWS), :], dst,
                                  sem).wait()

    def compute(sl):
        bufs = xv[sl]

        def row(r, c1):
            for j in range(LANE_GROUPS):
                s16 = pl.ds(j * 16, 16)
                e_val = ev[sl][r, s16]
                for b in range(BATCHES):
                    bufs[b][r, s16] = bufs[b][r, s16] + e_val
            return c1

        lax.fori_loop(0, GROUP_ROWS, row, 0)

    def group(e, sl, first_pair, last):
        other = 1 - sl
        if not first_pair:
            wait(osem[other], xv[other][0], BATCHES)
        if not last:
            issue_loads(e + 1, other)
        wait(esem[sl], ev[sl], 1)
        wait(isem[sl], xv[sl][0], BATCHES)
        compute(sl)
        issue_stores(e, sl)

    issue_loads(0, 0)
    group(0, 0, True, False)
    group(1, 1, True, False)

    def pair(k, carry):
        group(2 * k, 0, False, False)
        group(2 * k + 1, 1, False, False)
        return carry

    lax.fori_loop(1, NUM_GROUPS // 2 - 1, pair, 0)
    group(NUM_GROUPS - 2, 0, False, False)
    group(NUM_GROUPS - 1, 1, False, True)
    wait(osem[0], xv[0][0], BATCHES)
    wait(osem[1], xv[1][0], BATCHES)


def _sc_fill_tail(out_ref, x, emb):
    mesh = plsc.VectorSubcoreMesh(core_axis_name="c", subcore_axis_name="s")
    buf = pltpu.VMEM((GROUP_ROWS, D_MODEL), jnp.float32)
    pl.kernel(
        _sc_body,
        out_type=(),
        mesh=mesh,
        scratch_types=[buf] * 10 + [pltpu.SemaphoreType.DMA] * 6,
        compiler_params=pltpu.CompilerParams(use_tc_tiling_on_sc=True),
    )(out_ref, x, emb)


def _add_kernel(x_ref, emb_ref, o_ref):
    o_ref[...] = x_ref[...] + emb_ref[...]


def _tc_head(x, emb):
    bs, sl, d = x.shape
    return pl.pallas_call(
        _add_kernel,
        grid=(HEAD // SEQ_BLOCK,),
        in_specs=[
            pl.BlockSpec((bs, SEQ_BLOCK, d), lambda i: (0, i, 0)),
            pl.BlockSpec((SEQ_BLOCK, d), lambda i: (i, 0)),
        ],
        out_specs=pl.BlockSpec((bs, SEQ_BLOCK, d), lambda i: (0, i, 0)),
        out_shape=jax.ShapeDtypeStruct((bs, sl, d), x.dtype),
    )(x, emb)


def kernel(x, emb):
    head_full = _tc_head(x, emb)
    out_ref = jax.new_ref(head_full)
    _sc_fill_tail(out_ref, x, emb)
    return out_ref[...]
